# Initial kernel scaffold; baseline (speedup 1.0000x reference)
import jax
import jax.numpy as jnp
from jax.experimental import pallas as pl


def kernel(x, W, b):
    raise NotImplementedError("write your pallas kernel here")


# speedup vs baseline: 29.2295x; 29.2295x over previous
import jax
import jax.numpy as jnp
from jax.experimental import pallas as pl

N_A = 128
R_BLK = 256


def _tc_body(x_ref, wt_ref, b_ref, o_ref):
    xb = x_ref[...]
    R, L = xb.shape
    iota = jax.lax.broadcasted_iota(jnp.int32, (R, N_A), 1)
    hist = jnp.zeros((R, N_A), jnp.float32)
    for l in range(1, L):
        col = jax.lax.slice(xb, (0, l), (R, l + 1))
        hist = hist + (col == iota).astype(jnp.float32)
    q = jnp.dot(hist, wt_ref[...], preferred_element_type=jnp.float32)
    o_ref[...] = q + b_ref[...]


def kernel(x, W, b):
    B, L = x.shape
    Wt = W.T
    b2 = b.reshape(1, N_A)
    grid = (B // R_BLK,)
    return pl.pallas_call(
        _tc_body,
        grid=grid,
        in_specs=[
            pl.BlockSpec((R_BLK, L), lambda i: (i, 0)),
            pl.BlockSpec((N_A, N_A), lambda i: (0, 0)),
            pl.BlockSpec((1, N_A), lambda i: (0, 0)),
        ],
        out_specs=pl.BlockSpec((R_BLK, N_A), lambda i: (i, 0)),
        out_shape=jax.ShapeDtypeStruct((B, N_A), jnp.float32),
    )(x, Wt, b2)


# speedup vs baseline: 89.6570x; 3.0673x over previous
import functools
import jax
import jax.numpy as jnp
from jax import lax
from jax.experimental import pallas as pl
from jax.experimental.pallas import tpu as pltpu
from jax.experimental.pallas import tpu_sc as plsc

N_A = 128
B = 4096
HIST_LEN = 200
NW = 32
ROWS = B // NW
LANES = 16
NGRP = 13


def _sc_hist(x_hbm, out_hbm, x_v, hist_v):
    wid = lax.axis_index("s") * 2 + lax.axis_index("c")
    base = wid * ROWS
    pltpu.sync_copy(x_hbm.at[pl.ds(base, ROWS)], x_v)

    lane = lax.iota(jnp.int32, LANES)
    mask_first = lane >= 1
    mask_last = lane >= 8
    ones = jnp.ones((LANES,), jnp.float32)
    zeros = jnp.zeros((LANES,), jnp.float32)

    def row_body(r, carry):
        hbase = r * N_A
        for h in range(N_A // LANES):
            hist_v[pl.ds(hbase + h * LANES, LANES)] = zeros
        row_off = jnp.full((LANES,), hbase, jnp.int32)
        for g in range(NGRP):
            off = g * LANES if g < NGRP - 1 else HIST_LEN - LANES
            bins = x_v[r, pl.ds(off, LANES)] + row_off
            if g == 0:
                m = mask_first
            elif g == NGRP - 1:
                m = mask_last
            else:
                m = None
            plsc.addupdate_scatter(hist_v, [bins], ones, mask=m)
        return carry

    lax.fori_loop(0, ROWS, row_body, 0)
    pltpu.sync_copy(hist_v, out_hbm.at[pl.ds(base * N_A, ROWS * N_A)])


@jax.jit
def _hist_sc(x):
    mesh = plsc.VectorSubcoreMesh(core_axis_name="c", subcore_axis_name="s")
    f = functools.partial(
        pl.kernel,
        mesh=mesh,
        out_type=jax.ShapeDtypeStruct((B * N_A,), jnp.float32),
        scratch_types=[
            pltpu.VMEM((ROWS, HIST_LEN), jnp.int32),
            pltpu.VMEM((ROWS * N_A,), jnp.float32),
        ],
        compiler_params=pltpu.CompilerParams(needs_layout_passes=False),
    )(_sc_hist)
    return f(x).reshape(B, N_A)


def _tc_body(h_ref, wt_ref, b_ref, o_ref):
    q = jnp.dot(h_ref[...], wt_ref[...], preferred_element_type=jnp.float32)
    o_ref[...] = q + b_ref[...]


def _matmul_tc(hist, Wt, b2):
    R = 512
    return pl.pallas_call(
        _tc_body,
        grid=(B // R,),
        in_specs=[
            pl.BlockSpec((R, N_A), lambda i: (i, 0)),
            pl.BlockSpec((N_A, N_A), lambda i: (0, 0)),
            pl.BlockSpec((1, N_A), lambda i: (0, 0)),
        ],
        out_specs=pl.BlockSpec((R, N_A), lambda i: (i, 0)),
        out_shape=jax.ShapeDtypeStruct((B, N_A), jnp.float32),
    )(hist, Wt, b2)


def kernel(x, W, b):
    hist = _hist_sc(x)
    return _matmul_tc(hist, W.T, b.reshape(1, N_A))


# speedup vs baseline: 112.4793x; 1.2546x over previous
import functools
import jax
import jax.numpy as jnp
from jax import lax
from jax.experimental import pallas as pl
from jax.experimental.pallas import tpu as pltpu
from jax.experimental.pallas import tpu_sc as plsc

N_A = 128
B = 4096
HIST_LEN = 200
NW = 32
ROWS = B // NW
LANES = 16
NGRP = 13


def _sc_hist(x_hbm, out_hbm, x_v, hist_v):
    wid = lax.axis_index("s") * 2 + lax.axis_index("c")
    base = wid * ROWS
    pltpu.sync_copy(x_hbm.at[pl.ds(base, ROWS)], x_v)

    lane = lax.iota(jnp.int32, LANES)
    mask_first = lane >= 1
    mask_last = lane >= 8
    ones = jnp.ones((LANES,), jnp.float32)
    zeros = jnp.zeros((LANES,), jnp.float32)

    @plsc.parallel_loop(0, ROWS, unroll=4)
    def row_body(r):
        hbase = r * N_A
        for h in range(N_A // LANES):
            hist_v[pl.ds(hbase + h * LANES, LANES)] = zeros
        row_off = jnp.full((LANES,), hbase, jnp.int32)
        for g in range(NGRP):
            off = g * LANES if g < NGRP - 1 else HIST_LEN - LANES
            bins = x_v[r, pl.ds(off, LANES)] + row_off
            if g == 0:
                m = mask_first
            elif g == NGRP - 1:
                m = mask_last
            else:
                m = None
            plsc.addupdate_scatter(hist_v, [bins], ones, mask=m)
    pltpu.sync_copy(hist_v, out_hbm.at[pl.ds(base * N_A, ROWS * N_A)])


@jax.jit
def _hist_sc(x):
    mesh = plsc.VectorSubcoreMesh(core_axis_name="c", subcore_axis_name="s")
    f = functools.partial(
        pl.kernel,
        mesh=mesh,
        out_type=jax.ShapeDtypeStruct((B * N_A,), jnp.float32),
        scratch_types=[
            pltpu.VMEM((ROWS, HIST_LEN), jnp.int32),
            pltpu.VMEM((ROWS * N_A,), jnp.float32),
        ],
        compiler_params=pltpu.CompilerParams(needs_layout_passes=False),
    )(_sc_hist)
    return f(x).reshape(B, N_A)


def _tc_body(h_ref, wt_ref, b_ref, o_ref):
    q = jnp.dot(h_ref[...], wt_ref[...], preferred_element_type=jnp.float32)
    o_ref[...] = q + b_ref[...]


def _matmul_tc(hist, Wt, b2):
    R = 512
    return pl.pallas_call(
        _tc_body,
        grid=(B // R,),
        in_specs=[
            pl.BlockSpec((R, N_A), lambda i: (i, 0)),
            pl.BlockSpec((N_A, N_A), lambda i: (0, 0)),
            pl.BlockSpec((1, N_A), lambda i: (0, 0)),
        ],
        out_specs=pl.BlockSpec((R, N_A), lambda i: (i, 0)),
        out_shape=jax.ShapeDtypeStruct((B, N_A), jnp.float32),
    )(hist, Wt, b2)


def kernel(x, W, b):
    hist = _hist_sc(x)
    return _matmul_tc(hist, W.T, b.reshape(1, N_A))
